# baseline (speedup 1.0000x reference)
import functools

import jax
import jax.numpy as jnp
from jax import lax
from jax.experimental import pallas as pl
from jax.experimental.pallas import tpu as pltpu
from jax.experimental.pallas import tpu_sc as plsc

_BATCH = 16384
_EMB = 32
_NTAB = 4
_VOCAB = 100000
_ROWW = 128
_VPR = _ROWW // _EMB
_PROWS = _VOCAB // _VPR
_NCHUNKS_PACK = _VOCAB // _ROWW
_TAIL0 = _NCHUNKS_PACK * _ROWW
_NC = 2
_NS = 16
_NW = _NC * _NS
_BPW = _BATCH // _NW
_CHUNK = 128
_NCHUNK = _BPW // _CHUNK

_mesh = plsc.VectorSubcoreMesh(core_axis_name="c", subcore_axis_name="s")

_BLK = 512
_BROWS = _BLK // _VPR
_NBLOCKS = _VOCAB // _BLK
_MAINM = 24


def _make_pack_kernel():
  p_ty = jax.ShapeDtypeStruct((_PROWS, _ROWW), jnp.float32)

  @functools.partial(
      pl.kernel,
      mesh=_mesh,
      out_type=(p_ty,) * _NTAB,
      compiler_params=pltpu.CompilerParams(needs_layout_passes=False),
      scratch_types=[
          pltpu.VMEM((3, _EMB, _BLK), jnp.float32),
          pltpu.VMEM((2, _BROWS, _ROWW), jnp.float32),
          pltpu.SemaphoreType.DMA,
          pltpu.SemaphoreType.DMA,
      ],
  )
  def body(wgT, waT, woT, wzT, tails, pg, pa, po, pz, in_v, out_v,
           sin, sout):
    wid = lax.axis_index("s") * _NC + lax.axis_index("c")
    tt_dyn = wid % _NTAB
    slot = wid // _NTAB
    iota = lax.iota(jnp.int32, 16)
    tabs = (wgT, waT, woT, wzT)
    packs = (pg, pa, po, pz)


    rows8 = [(iota + 16 * j) >> 2 for j in range(_ROWW // 16)]
    colb8 = [((iota + 16 * j) & 3) * _EMB for j in range(_ROWW // 16)]

    def transpose_sub(src, dst, q, nq=_BLK // _ROWW):
      for e in range(_EMB):
        vs = [src[e, pl.ds(q * _ROWW + j * 16, 16)]
              for j in range(_ROWW // 16)]
        for j in range(_ROWW // 16):
          plsc.store_scatter(dst, [rows8[j] + 32 * q, colb8[j] + e], vs[j])

    def transpose_block(src, dst):
      for q in range(_BLK // _ROWW):
        transpose_sub(src, dst, q)

    def in_start(b, buf, cols=_BLK):
      for tt in range(_NTAB):
        @pl.when(tt_dyn == tt)
        def _(tt=tt):
          pltpu.make_async_copy(
              tabs[tt].at[:, pl.ds(b * _BLK, cols)],
              in_v.at[buf, :, pl.ds(0, cols)], sin).start()

    def in_wait(buf, cols=_BLK):
      pltpu.make_async_copy(
          tabs[0].at[:, pl.ds(0, cols)],
          in_v.at[buf, :, pl.ds(0, cols)], sin).wait()

    def out_start(b, buf, rows=_BROWS):
      for tt in range(_NTAB):
        @pl.when(tt_dyn == tt)
        def _(tt=tt):
          pltpu.make_async_copy(
              out_v.at[buf, pl.ds(0, rows)],
              packs[tt].at[pl.ds(b * _BROWS, rows)], sout).start()

    def out_wait(buf, rows=_BROWS):
      pltpu.make_async_copy(
          out_v.at[buf, pl.ds(0, rows)],
          packs[0].at[pl.ds(0, rows)], sout).wait()

    for p in range(3):
      in_start(slot + 8 * p, p)

    def step(m, _):
      in_wait(m % 3)

      @pl.when(m >= 2)
      def _():
        out_wait(m % 2)

      transpose_block(in_v.at[m % 3], out_v.at[m % 2])
      out_start(slot + 8 * m, m % 2)

      @pl.when(m + 3 < _MAINM)
      def _():
        in_start(slot + 8 * (m + 3), (m + 3) % 3)
      return ()

    lax.fori_loop(0, _MAINM, step, ())
    out_wait(0)
    out_wait(1)

    @pl.when(slot < _NBLOCKS - 8 * _MAINM)
    def _():
      in_start(8 * _MAINM + slot, 0)
      in_wait(0)
      transpose_block(in_v.at[0], out_v.at[0])
      out_start(8 * _MAINM + slot, 0)
      out_wait(0)

    @pl.when(slot == 3)
    def _():
      for tt in range(_NTAB):
        @pl.when(tt_dyn == tt)
        def _(tt=tt):
          pltpu.make_async_copy(
              tabs[tt].at[:, pl.ds(_NBLOCKS * _BLK, _ROWW)],
              in_v.at[0, :, pl.ds(0, _ROWW)], sin).start()
      in_wait(0, cols=_ROWW)
      transpose_sub(in_v.at[0], out_v.at[0], 0)
      for tt in range(_NTAB):
        @pl.when(tt_dyn == tt)
        def _(tt=tt):
          pltpu.make_async_copy(
              out_v.at[0, pl.ds(0, _EMB)],
              packs[tt].at[pl.ds(_NBLOCKS * _BROWS, _EMB)], sout).start()
      out_wait(0, rows=_EMB)

    @pl.when(slot == 4)
    def _():
      for tt in range(_NTAB):
        @pl.when(tt_dyn == tt)
        def _(tt=tt):
          pltpu.sync_copy(tails.at[tt], out_v.at[0, pl.ds(0, 8)])
          pltpu.sync_copy(out_v.at[0, pl.ds(0, 8)],
                          packs[tt].at[pl.ds(_TAIL0 // _VPR, 8)])

  return body


def _make_gather_kernel():
  @functools.partial(
      pl.kernel,
      mesh=_mesh,
      out_type=jax.ShapeDtypeStruct((_BATCH, _NTAB * _EMB), jnp.float32),
      scratch_types=[
          pltpu.VMEM((_NTAB, _CHUNK), jnp.int32),
          pltpu.VMEM((_NTAB, _CHUNK), jnp.int32),
          pltpu.VMEM((_NTAB, _CHUNK, _ROWW), jnp.float32),
          pltpu.VMEM((_CHUNK, _NTAB * _EMB), jnp.float32),
          pltpu.SemaphoreType.DMA,
      ],
  )
  def body(xT, pg, pa, po, pz, out_hbm, idx_v, q_v, rows_v, out_v, gsem):
    wid = lax.axis_index("s") * _NC + lax.axis_index("c")
    base = wid * _BPW
    tables = (pg, pa, po, pz)
    for j in range(_NCHUNK):
      b0 = base + j * _CHUNK
      pltpu.sync_copy(xT.at[:, pl.ds(b0, _CHUNK)], idx_v)
      for t in range(_NTAB):
        for v in range(_CHUNK // 16):
          q_v[t, pl.ds(v * 16, 16)] = jax.lax.shift_right_logical(
              idx_v[t, pl.ds(v * 16, 16)], 2)
      copies = [
          pltpu.async_copy(tables[t].at[q_v.at[t]], rows_v.at[t], gsem)
          for t in range(_NTAB)
      ]
      for cp in copies:
        cp.wait()

      def select(g, _):
        for t in range(_NTAB):
          iv = idx_v[t, pl.ds(g * 16, 16)]
          for l in range(16):
            off = (iv[l] & (_VPR - 1)) * _EMB
            b = g * 16 + l
            for k in range(_EMB // 16):
              out_v[b, pl.ds(t * _EMB + k * 16, 16)] = (
                  rows_v[t, b, pl.ds(off + k * 16, 16)])
        return ()

      lax.fori_loop(0, _CHUNK // 16, select, ())
      pltpu.sync_copy(out_v, out_hbm.at[pl.ds(b0, _CHUNK)])

  return body


_pack = _make_pack_kernel()
_gather = _make_gather_kernel()


def kernel(x, W_gender, W_age, W_occupation, W_zip):
  ws = (W_gender, W_age, W_occupation, W_zip)
  tails = jnp.stack([w[_TAIL0:].reshape(8, _ROWW) for w in ws])
  packed = _pack(*[w.T for w in ws], tails)
  return _gather(x.T, *packed)


# speedup vs baseline: 1.7366x; 1.7366x over previous
import functools

import jax
import jax.numpy as jnp
from jax import lax
from jax.experimental import pallas as pl
from jax.experimental.pallas import tpu as pltpu
from jax.experimental.pallas import tpu_sc as plsc

_BATCH = 16384
_EMB = 32
_NTAB = 4
_ROWW = _NTAB * _EMB
_NC = 2
_NS = 16
_NW = _NC * _NS
_BPW = _BATCH // _NW
_CHUNK = 128
_NCHUNK = _BPW // _CHUNK

_mesh = plsc.VectorSubcoreMesh(core_axis_name="c", subcore_axis_name="s")


def _make_gather_kernel():
  @functools.partial(
      pl.kernel,
      mesh=_mesh,
      out_type=jax.ShapeDtypeStruct((_BATCH, _ROWW), jnp.float32),
      scratch_types=[
          pltpu.VMEM((_NTAB, _CHUNK), jnp.int32),
          pltpu.VMEM((_NTAB, _CHUNK, _ROWW), jnp.float32),
          pltpu.VMEM((_CHUNK, _ROWW), jnp.float32),
          pltpu.SemaphoreType.DMA,
      ],
  )
  def body(xT, wall, out_hbm, idx_v, rows_v, out_v, gsem):
    wid = lax.axis_index("s") * _NC + lax.axis_index("c")
    base = wid * _BPW
    for j in range(_NCHUNK):
      b0 = base + j * _CHUNK
      pltpu.sync_copy(xT.at[:, pl.ds(b0, _CHUNK)], idx_v)
      copies = [
          pltpu.async_copy(wall.at[idx_v.at[t]], rows_v.at[t], gsem)
          for t in range(_NTAB)
      ]
      for cp in copies:
        cp.wait()

      def select(b, _):
        for t in range(_NTAB):
          for k in range(_EMB // 16):
            c = t * _EMB + k * 16
            out_v[b, pl.ds(c, 16)] = rows_v[t, b, pl.ds(c, 16)]
        return ()

      lax.fori_loop(0, _CHUNK, select, ())
      pltpu.sync_copy(out_v, out_hbm.at[pl.ds(b0, _CHUNK)])

  return body


_gather = _make_gather_kernel()


def kernel(x, W_gender, W_age, W_occupation, W_zip):
  w_all = jnp.concatenate((W_gender, W_age, W_occupation, W_zip), axis=1)
  return _gather(x.T, w_all)
